# Initial kernel scaffold; baseline (speedup 1.0000x reference)
import jax
import jax.numpy as jnp
from jax.experimental import pallas as pl


def kernel(x, edge_index, W1, a_s1, a_d1, b1, W2, a_s2, a_d2, b2):
    raise NotImplementedError("write your pallas kernel here")


# speedup vs baseline: 26.9313x; 26.9313x over previous
import functools

import jax
import jax.numpy as jnp
from jax import lax
from jax.experimental import pallas as pl
from jax.experimental.pallas import tpu as pltpu
from jax.experimental.pallas import tpu_sc as plsc

NC = 2
NS = 16
NW = NC * NS
LANES = 16
CHUNK = 128
ROWS_PIECE = 128


def _proj_body(x_ref, w_ref, as_w_ref, ad_w_ref, h_ref, a1_ref, a2_ref,
               *, d, w_pad):
    h = lax.dot_general(x_ref[...], w_ref[...], (((1,), (0,)), ((), ())),
                        precision=lax.Precision.HIGHEST,
                        preferred_element_type=jnp.float32)
    h_ref[:, :d] = h
    if w_pad > d:
        h_ref[:, d:] = jnp.zeros((h.shape[0], w_pad - d), jnp.float32)
    a1_ref[...] = jnp.sum(h * as_w_ref[...], axis=1, keepdims=True)
    a2_ref[...] = jnp.sum(h * ad_w_ref[...], axis=1, keepdims=True)


def _proj(x, w, as_w, ad_w, w_pad, rows_blk):
    n, _ = x.shape
    d = w.shape[1]
    grid = (n // rows_blk,)
    return pl.pallas_call(
        functools.partial(_proj_body, d=d, w_pad=w_pad),
        grid=grid,
        in_specs=[
            pl.BlockSpec((rows_blk, x.shape[1]), lambda i: (i, 0)),
            pl.BlockSpec((w.shape[0], d), lambda i: (0, 0)),
            pl.BlockSpec((1, d), lambda i: (0, 0)),
            pl.BlockSpec((1, d), lambda i: (0, 0)),
        ],
        out_specs=[
            pl.BlockSpec((rows_blk, w_pad), lambda i: (i, 0)),
            pl.BlockSpec((rows_blk, 1), lambda i: (i, 0)),
            pl.BlockSpec((rows_blk, 1), lambda i: (i, 0)),
        ],
        out_shape=[
            jax.ShapeDtypeStruct((n, w_pad), jnp.float32),
            jax.ShapeDtypeStruct((n, 1), jnp.float32),
            jax.ShapeDtypeStruct((n, 1), jnp.float32),
        ],
    )(x, w, as_w, ad_w)


def _mid_body(acc_ref, b_ref, w_ref, as_w_ref, ad_w_ref,
              h_ref, a1_ref, a2_ref, *, d_prev, d, w_pad):
    s = acc_ref[0] + acc_ref[1]
    o = s[:, :d_prev] / (s[:, d_prev:d_prev + 1] + 1e-16) + b_ref[...]
    hin = jnp.maximum(o, 0.0)
    h = lax.dot_general(hin, w_ref[...], (((1,), (0,)), ((), ())),
                        precision=lax.Precision.HIGHEST,
                        preferred_element_type=jnp.float32)
    h_ref[:, :d] = h
    if w_pad > d:
        h_ref[:, d:] = jnp.zeros((h.shape[0], w_pad - d), jnp.float32)
    a1_ref[...] = jnp.sum(h * as_w_ref[...], axis=1, keepdims=True)
    a2_ref[...] = jnp.sum(h * ad_w_ref[...], axis=1, keepdims=True)


def _mid(acc, b, w, as_w, ad_w, w_pad, rows_blk, n):
    wp_prev = acc.shape[2]
    d_prev = b.shape[1]
    d = w.shape[1]
    grid = (n // rows_blk,)
    return pl.pallas_call(
        functools.partial(_mid_body, d_prev=d_prev, d=d, w_pad=w_pad),
        grid=grid,
        in_specs=[
            pl.BlockSpec((NC, rows_blk, wp_prev), lambda i: (0, i, 0)),
            pl.BlockSpec((1, d_prev), lambda i: (0, 0)),
            pl.BlockSpec((w.shape[0], d), lambda i: (0, 0)),
            pl.BlockSpec((1, d), lambda i: (0, 0)),
            pl.BlockSpec((1, d), lambda i: (0, 0)),
        ],
        out_specs=[
            pl.BlockSpec((rows_blk, w_pad), lambda i: (i, 0)),
            pl.BlockSpec((rows_blk, 1), lambda i: (i, 0)),
            pl.BlockSpec((rows_blk, 1), lambda i: (i, 0)),
        ],
        out_shape=[
            jax.ShapeDtypeStruct((n, w_pad), jnp.float32),
            jax.ShapeDtypeStruct((n, 1), jnp.float32),
            jax.ShapeDtypeStruct((n, 1), jnp.float32),
        ],
    )(acc, b, w, as_w, ad_w)


def _fin_body(acc_ref, b_ref, o_ref, *, d_prev):
    s = acc_ref[0] + acc_ref[1]
    o = s[:, :d_prev] / (s[:, d_prev:d_prev + 1] + 1e-16) + b_ref[...]
    m = jnp.max(o, axis=1, keepdims=True)
    ex = jnp.exp(o - m)
    lse = jnp.log(jnp.sum(ex, axis=1, keepdims=True))
    o_ref[...] = o - m - lse


def _fin(acc, b, rows_blk, n):
    wp_prev = acc.shape[2]
    d_prev = b.shape[1]
    grid = (n // rows_blk,)
    return pl.pallas_call(
        functools.partial(_fin_body, d_prev=d_prev),
        grid=grid,
        in_specs=[
            pl.BlockSpec((NC, rows_blk, wp_prev), lambda i: (0, i, 0)),
            pl.BlockSpec((1, d_prev), lambda i: (0, 0)),
        ],
        out_specs=pl.BlockSpec((rows_blk, d_prev), lambda i: (i, 0)),
        out_shape=jax.ShapeDtypeStruct((n, d_prev), jnp.float32),
    )(acc, b)


def _make_edge_pass(n, e_pad, e_real, d, w_pad):
    per_w = e_pad // NW
    n_chunks = per_w // CHUNK
    n_acc = -(-n // (NS * ROWS_PIECE)) * (NS * ROWS_PIECE)
    rows_pt = n_acc // NS
    n_pieces = rows_pt // ROWS_PIECE
    nsb = -(-d // LANES)
    mesh = plsc.VectorSubcoreMesh(core_axis_name="c", subcore_axis_name="s")

    @functools.partial(
        pl.kernel,
        out_type=jax.ShapeDtypeStruct((NC, n_acc, w_pad), jnp.float32),
        mesh=mesh,
        compiler_params=pltpu.CompilerParams(needs_layout_passes=False,
                                             use_tc_tiling_on_sc=False),
        scratch_types=[
            pltpu.VMEM((CHUNK,), jnp.int32),
            pltpu.VMEM((CHUNK,), jnp.int32),
            pltpu.VMEM((CHUNK, w_pad), jnp.float32),
            pltpu.VMEM((n,), jnp.float32),
            pltpu.VMEM((n,), jnp.float32),
            pltpu.VMEM_SHARED((n_acc, w_pad), jnp.float32),
        ],
    )
    def edge_kernel(src_hbm, dst_hbm, as_hbm, ad_hbm, h_hbm, out_hbm,
                    src_v, dst_v, rows_v, as_v, ad_v, acc_s):
        c = lax.axis_index("c")
        s = lax.axis_index("s")
        wid = c * NS + s
        iota = lax.iota(jnp.int32, LANES)
        zeros16 = jnp.zeros((LANES,), jnp.float32)

        @pl.loop(0, ROWS_PIECE)
        def _(i):
            for q in range(w_pad // LANES):
                rows_v[i, pl.ds(q * LANES, LANES)] = zeros16

        @pl.loop(0, n_pieces)
        def _(p):
            pltpu.sync_copy(
                rows_v, acc_s.at[pl.ds(s * rows_pt + p * ROWS_PIECE, ROWS_PIECE)])

        pltpu.sync_copy(as_hbm, as_v)
        pltpu.sync_copy(ad_hbm, ad_v)
        plsc.subcore_barrier()

        col_d = jnp.full((LANES,), d, jnp.int32)

        @pl.loop(0, n_chunks)
        def _(j):
            off = wid * per_w + j * CHUNK
            pltpu.sync_copy(src_hbm.at[pl.ds(off, CHUNK)], src_v)
            pltpu.sync_copy(dst_hbm.at[pl.ds(off, CHUNK)], dst_v)
            pltpu.sync_copy(h_hbm.at[src_v], rows_v)

            @pl.loop(0, CHUNK // LANES)
            def _(g):
                si = src_v[pl.ds(g * LANES, LANES)]
                di = dst_v[pl.ds(g * LANES, LANES)]
                a = plsc.load_gather(as_v, [si]) + plsc.load_gather(ad_v, [di])
                a = jnp.where(a > 0, a, 0.2 * a)
                ev = jnp.exp(a)
                ev = jnp.where(off + g * LANES + iota < e_real, ev, 0.0)
                for l in range(LANES):
                    es16 = jnp.full((LANES,), ev[l], jnp.float32)
                    row = g * LANES + l
                    for q in range(nsb):
                        rows_v[row, pl.ds(q * LANES, LANES)] = (
                            rows_v[row, pl.ds(q * LANES, LANES)] * es16)
                plsc.store_scatter(rows_v, [g * LANES + iota, col_d], ev)

            pltpu.sync_copy(rows_v, acc_s.at[dst_v], add=True)

        plsc.subcore_barrier()

        @pl.loop(0, n_pieces)
        def _(p):
            r0 = s * rows_pt + p * ROWS_PIECE
            pltpu.sync_copy(acc_s.at[pl.ds(r0, ROWS_PIECE)], rows_v)
            pltpu.sync_copy(rows_v, out_hbm.at[c, pl.ds(r0, ROWS_PIECE)])

    return edge_kernel


def kernel(x, edge_index, W1, a_s1, a_d1, b1, W2, a_s2, a_d2, b2):
    n, _ = x.shape
    e = edge_index.shape[1]
    e_tot = e + n
    e_pad = -(-e_tot // (NW * CHUNK)) * (NW * CHUNK)

    loops = jnp.arange(n, dtype=edge_index.dtype)
    padz = jnp.zeros((e_pad - e_tot,), edge_index.dtype)
    src = jnp.concatenate([edge_index[0], loops, padz])
    dst = jnp.concatenate([edge_index[1], loops, padz])

    h1p, as1, ad1 = _proj(x, W1, a_s1.reshape(1, -1), a_d1.reshape(1, -1),
                          w_pad=144, rows_blk=2000)
    acc1 = _make_edge_pass(n, e_pad, e_tot, d=128, w_pad=144)(
        src, dst, as1.reshape(-1), ad1.reshape(-1), h1p)
    h2p, as2, ad2 = _mid(acc1, b1.reshape(1, -1), W2,
                         a_s2.reshape(1, -1), a_d2.reshape(1, -1),
                         w_pad=48, rows_blk=2000, n=n)
    acc2 = _make_edge_pass(n, e_pad, e_tot, d=40, w_pad=48)(
        src, dst, as2.reshape(-1), ad2.reshape(-1), h2p)
    return _fin(acc2, b2.reshape(1, -1), rows_blk=2000, n=n)
